# baseline (speedup 1.0000x reference)
import functools
import jax
import jax.numpy as jnp
from jax import lax
from jax.experimental import pallas as pl

N = 10000
E = 320000
T = 640000
A = 192
EMB = 128
TRIP = 64
RBF = 128
SPH = 7
NB = 3
CUT = 6.0

NPAD = 10240

EB = 1000
TB = 2000
NBLK = 1000


def _silu(x):
    return x * jax.lax.logistic(x)


def _node_init_kern(z_ref, lat_ref, tab_ref, wl1_ref, wl2_ref, h_ref):
    z = z_ref[...]
    ids = jax.lax.broadcasted_iota(jnp.int32, (z.shape[0], 128), 1)
    onehot = (z == ids).astype(jnp.float32)
    th = onehot @ tab_ref[...]
    x = jnp.concatenate([th, lat_ref[...]], axis=1)
    h_ref[...] = jax.nn.relu(x @ wl1_ref[...]) @ wl2_ref[...]


def _node_init(z, latent, atom_table, Wl1, Wl2):
    z2 = z.astype(jnp.int32).reshape(N, 1)
    tab = jnp.zeros((128, 128), jnp.float32).at[:100, :].set(atom_table)
    grid = N // NBLK
    return pl.pallas_call(
        _node_init_kern,
        grid=(grid,),
        in_specs=[
            pl.BlockSpec((NBLK, 1), lambda i: (i, 0)),
            pl.BlockSpec((NBLK, 64), lambda i: (i, 0)),
            pl.BlockSpec((128, 128), lambda i: (0, 0)),
            pl.BlockSpec((192, 192), lambda i: (0, 0)),
            pl.BlockSpec((192, 192), lambda i: (0, 0)),
        ],
        out_specs=pl.BlockSpec((NBLK, 192), lambda i: (i, 0)),
        out_shape=jax.ShapeDtypeStruct((N, 192), jnp.float32),
    )(z2, latent, tab, Wl1, Wl2)


def _edge_kern(ps_ref, pt_ref, hs_ref, ht_ref, we_ref, wr_ref,
               m_ref, unit_ref, rbfp_ref):
    vec = pt_ref[...] - ps_ref[...]
    d2 = jnp.sum(vec * vec, axis=1) + 1e-8
    D = jnp.sqrt(d2)
    unit = vec / D[:, None]
    dsc = D / CUT
    d5 = dsc * dsc * dsc * dsc * dsc
    env = jnp.where(
        dsc < 1.0,
        1.0 - 21.0 * d5 + 35.0 * d5 * dsc - 15.0 * d5 * dsc * dsc,
        0.0,
    )
    B = vec.shape[0]
    centers = (jax.lax.broadcasted_iota(jnp.int32, (B, RBF), 1)
               .astype(jnp.float32) * (CUT / (RBF - 1)))
    gamma = (RBF / CUT) ** 2
    rbf = env[:, None] * jnp.exp(-gamma * (D[:, None] - centers) ** 2)
    x = jnp.concatenate([hs_ref[...], ht_ref[...], rbf], axis=1)
    m_ref[...] = _silu(x @ we_ref[...])
    unit_ref[...] = unit
    rbfp_ref[...] = rbf @ wr_ref[...]


def _edge_stage(pos_s, pos_t, h_s, h_t, W_edge, W_rbf_all):
    grid = E // EB
    return pl.pallas_call(
        _edge_kern,
        grid=(grid,),
        in_specs=[
            pl.BlockSpec((EB, 16), lambda i: (i, 0)),
            pl.BlockSpec((EB, 16), lambda i: (i, 0)),
            pl.BlockSpec((EB, 192), lambda i: (i, 0)),
            pl.BlockSpec((EB, 192), lambda i: (i, 0)),
            pl.BlockSpec((512, 128), lambda i: (0, 0)),
            pl.BlockSpec((128, 64), lambda i: (0, 0)),
        ],
        out_specs=[
            pl.BlockSpec((EB, 128), lambda i: (i, 0)),
            pl.BlockSpec((EB, 16), lambda i: (i, 0)),
            pl.BlockSpec((EB, 64), lambda i: (i, 0)),
        ],
        out_shape=[
            jax.ShapeDtypeStruct((E, 128), jnp.float32),
            jax.ShapeDtypeStruct((E, 16), jnp.float32),
            jax.ShapeDtypeStruct((E, 64), jnp.float32),
        ],
    )(pos_s, pos_t, h_s, h_t, W_edge, W_rbf_all)


def _cbf_kern(uba_ref, uca_ref, rca_ref, wsph_ref, wup_ref, cbf_ref):
    cos = jnp.clip(jnp.sum(uba_ref[...] * uca_ref[...], axis=1), -1.0, 1.0)
    wsph = wsph_ref[...]
    C = cos.shape[0]
    acc = jnp.zeros((C, 16), jnp.float32)
    cp = jnp.ones((C,), jnp.float32)
    for l in range(SPH):
        acc = acc + cp[:, None] * wsph[l][None, :]
        cp = cp * cos
    cbf_ref[...] = (rca_ref[...] * acc) @ wup_ref[...]


def _cbf_stage(unit_ba, unit_ca, rbfc_ca, W_sph, W_cbf_up):
    wsph = jnp.zeros((8, 16), jnp.float32).at[:SPH, :].set(W_sph)
    grid = T // TB
    return pl.pallas_call(
        _cbf_kern,
        grid=(grid,),
        in_specs=[
            pl.BlockSpec((TB, 16), lambda i: (i, 0)),
            pl.BlockSpec((TB, 16), lambda i: (i, 0)),
            pl.BlockSpec((TB, 16), lambda i: (i, 0)),
            pl.BlockSpec((8, 16), lambda i: (0, 0)),
            pl.BlockSpec((16, 64), lambda i: (0, 0)),
        ],
        out_specs=pl.BlockSpec((TB, 64), lambda i: (i, 0)),
        out_shape=jax.ShapeDtypeStruct((T, 64), jnp.float32),
    )(unit_ba, unit_ca, rbfc_ca, wsph, W_cbf_up)


def _down_kern(m_ref, w1_ref, wd_ref, xd_ref):
    x = _silu(m_ref[...] @ w1_ref[...])
    xd_ref[...] = _silu(x @ wd_ref[...])


def _down_stage(m, W1, Wdown):
    grid = E // EB
    return pl.pallas_call(
        _down_kern,
        grid=(grid,),
        in_specs=[
            pl.BlockSpec((EB, 128), lambda i: (i, 0)),
            pl.BlockSpec((128, 128), lambda i: (0, 0)),
            pl.BlockSpec((128, 64), lambda i: (0, 0)),
        ],
        out_specs=pl.BlockSpec((EB, 64), lambda i: (i, 0)),
        out_shape=jax.ShapeDtypeStruct((E, 64), jnp.float32),
    )(m, W1, Wdown)


def _x3_kern(xd_ref, cbf_ref, x3_ref):
    x3_ref[...] = xd_ref[...] * cbf_ref[...]


def _x3_stage(xd_ba, cbf):
    grid = T // TB
    return pl.pallas_call(
        _x3_kern,
        grid=(grid,),
        in_specs=[
            pl.BlockSpec((TB, 64), lambda i: (i, 0)),
            pl.BlockSpec((TB, 64), lambda i: (i, 0)),
        ],
        out_specs=pl.BlockSpec((TB, 64), lambda i: (i, 0)),
        out_shape=jax.ShapeDtypeStruct((T, 64), jnp.float32),
    )(xd_ba, cbf)


def _mup_kern(m_ref, agg_ref, rbfp_ref, wup_ref, wg_ref, wh_ref,
              mn_ref, am_ref):
    rbf3g = rbfp_ref[...][:, 0:16]
    rbfhg = rbfp_ref[...][:, 16:32]
    m = m_ref[...] + _silu(agg_ref[...] @ wup_ref[...]) * (rbf3g @ wg_ref[...])
    mn_ref[...] = m
    am_ref[...] = m * (rbfhg @ wh_ref[...])


def _mup_stage(m, agg, rbfp, Wup, Wg, Wh):
    grid = E // EB
    return pl.pallas_call(
        _mup_kern,
        grid=(grid,),
        in_specs=[
            pl.BlockSpec((EB, 128), lambda i: (i, 0)),
            pl.BlockSpec((EB, 64), lambda i: (i, 0)),
            pl.BlockSpec((EB, 64), lambda i: (i, 0)),
            pl.BlockSpec((64, 128), lambda i: (0, 0)),
            pl.BlockSpec((16, 128), lambda i: (0, 0)),
            pl.BlockSpec((16, 128), lambda i: (0, 0)),
        ],
        out_specs=[
            pl.BlockSpec((EB, 128), lambda i: (i, 0)),
            pl.BlockSpec((EB, 128), lambda i: (i, 0)),
        ],
        out_shape=[
            jax.ShapeDtypeStruct((E, 128), jnp.float32),
            jax.ShapeDtypeStruct((E, 128), jnp.float32),
        ],
    )(m, agg, rbfp, Wup, Wg, Wh)


def _hup_kern(h_ref, ha0_ref, ha1_ref, was_ref, hn_ref):
    ha = ha0_ref[0] + ha1_ref[0]
    hn_ref[...] = h_ref[...] + _silu(ha @ was_ref[...])


def _hup_stage(h, ha2, Was):
    grid = N // NBLK
    return pl.pallas_call(
        _hup_kern,
        grid=(grid,),
        in_specs=[
            pl.BlockSpec((NBLK, 192), lambda i: (i, 0)),
            pl.BlockSpec((1, NBLK, 128), lambda i: (0, i, 0)),
            pl.BlockSpec((1, NBLK, 128), lambda i: (1, i, 0)),
            pl.BlockSpec((128, 192), lambda i: (0, 0)),
        ],
        out_specs=pl.BlockSpec((NBLK, 192), lambda i: (i, 0)),
        out_shape=jax.ShapeDtypeStruct((N, 192), jnp.float32),
    )(h, ha2, ha2, Was)


def _eres_kern(m_ref, hs_ref, ht_ref, wes_ref, mn_ref):
    x = hs_ref[...] + ht_ref[...]
    mn_ref[...] = m_ref[...] + _silu(x @ wes_ref[...])


def _eres_stage(m, h_s, h_t, Wes):
    grid = E // EB
    return pl.pallas_call(
        _eres_kern,
        grid=(grid,),
        in_specs=[
            pl.BlockSpec((EB, 128), lambda i: (i, 0)),
            pl.BlockSpec((EB, 192), lambda i: (i, 0)),
            pl.BlockSpec((EB, 192), lambda i: (i, 0)),
            pl.BlockSpec((192, 128), lambda i: (0, 0)),
        ],
        out_specs=pl.BlockSpec((EB, 128), lambda i: (i, 0)),
        out_shape=jax.ShapeDtypeStruct((E, 128), jnp.float32),
    )(m, h_s, h_t, Wes)


def _fout_kern(m_ref, rbfp_ref, unit_ref, wf_ref, fu_ref):
    gate = rbfp_ref[...][:, 32:48] @ wf_ref[...]
    f = jnp.sum(m_ref[...] * gate, axis=1)
    fu_ref[...] = f[:, None] * unit_ref[...]


def _fout_stage(m, rbfp, unit, Wout_f):
    grid = E // EB
    return pl.pallas_call(
        _fout_kern,
        grid=(grid,),
        in_specs=[
            pl.BlockSpec((EB, 128), lambda i: (i, 0)),
            pl.BlockSpec((EB, 64), lambda i: (i, 0)),
            pl.BlockSpec((EB, 16), lambda i: (i, 0)),
            pl.BlockSpec((16, 128), lambda i: (0, 0)),
        ],
        out_specs=pl.BlockSpec((EB, 16), lambda i: (i, 0)),
        out_shape=jax.ShapeDtypeStruct((E, 16), jnp.float32),
    )(m, rbfp, unit, Wout_f)


def _nout_kern(h_ref, f0_ref, f1_ref, we_ref, ea_ref, f_ref):
    ea_ref[...] = h_ref[...] @ we_ref[...]
    f_ref[...] = f0_ref[0] + f1_ref[0]


def _nout_stage(h, F2, Wout_e):
    wep = jnp.zeros((192, 128), jnp.float32).at[:, 0:1].set(Wout_e)
    grid = N // NBLK
    return pl.pallas_call(
        _nout_kern,
        grid=(grid,),
        in_specs=[
            pl.BlockSpec((NBLK, 192), lambda i: (i, 0)),
            pl.BlockSpec((1, NBLK, 16), lambda i: (0, i, 0)),
            pl.BlockSpec((1, NBLK, 16), lambda i: (1, i, 0)),
            pl.BlockSpec((192, 128), lambda i: (0, 0)),
        ],
        out_specs=[
            pl.BlockSpec((NBLK, 128), lambda i: (i, 0)),
            pl.BlockSpec((NBLK, 16), lambda i: (i, 0)),
        ],
        out_shape=[
            jax.ShapeDtypeStruct((N, 128), jnp.float32),
            jax.ShapeDtypeStruct((N, 16), jnp.float32),
        ],
    )(h, F2, F2, wep)


def _gather(table, idx):
    return jnp.take(table, idx, axis=0)


def _segsum_n(rows, idx):
    s = jax.ops.segment_sum(rows, idx, num_segments=NPAD)
    return jnp.stack([s, jnp.zeros_like(s)], axis=0)


def _segsum_e(x3, id3_ca):
    return jax.ops.segment_sum(x3, id3_ca, num_segments=E)


def kernel(pos, latent, atom_table, Wl1, Wl2, W_edge, W_rbf3, W_rbf_h,
           W_rbf_out, W_rbf_cbf, W_sph, W_cbf_up, W1s, Wdowns, Wups, Wgs,
           Whs, Was, Wes, Wout_e, Wout_f, z, edge_index, id3_ba, id3_ca):
    idx_s = edge_index[0].astype(jnp.int32)
    idx_t = edge_index[1].astype(jnp.int32)
    ba = id3_ba.astype(jnp.int32)
    ca = id3_ca.astype(jnp.int32)

    pos16 = jnp.zeros((N, 16), jnp.float32).at[:, :3].set(pos)
    W_rbf_all = jnp.concatenate([W_rbf3, W_rbf_h, W_rbf_out, W_rbf_cbf], axis=1)

    h = _node_init(z, latent, atom_table, Wl1, Wl2)

    pos_s = _gather(pos16, idx_s)
    pos_t = _gather(pos16, idx_t)
    h_s = _gather(h, idx_s)
    h_t = _gather(h, idx_t)
    m, unit, rbfp = _edge_stage(pos_s, pos_t, h_s, h_t, W_edge, W_rbf_all)

    rbfc = rbfp[:, 48:64] + 0.0
    unit_ba = _gather(unit, ba)
    unit_ca = _gather(unit, ca)
    rbfc_ca = _gather(rbfc, ca)
    cbf = _cbf_stage(unit_ba, unit_ca, rbfc_ca, W_sph, W_cbf_up)

    for b in range(NB):
        xd = _down_stage(m, W1s[b], Wdowns[b])
        xd_ba = _gather(xd, ba)
        x3 = _x3_stage(xd_ba, cbf)
        agg = _segsum_e(x3, ca)
        m, am = _mup_stage(m, agg, rbfp, Wups[b], Wgs[b], Whs[b])
        ha2 = _segsum_n(am, idx_t)
        h = _hup_stage(h, ha2, Was[b])
        h_s = _gather(h, idx_s)
        h_t = _gather(h, idx_t)
        m = _eres_stage(m, h_s, h_t, Wes[b])

    fu = _fout_stage(m, rbfp, unit, Wout_f)
    F2 = _segsum_n(fu, idx_t)
    ea, F = _nout_stage(h, F2, Wout_e)
    return jnp.concatenate([ea[:, 0:1], F[:, 0:3]], axis=1)


# speedup vs baseline: 5.5956x; 5.5956x over previous
import functools
import jax
import jax.numpy as jnp
from jax import lax
from jax.experimental import pallas as pl
from jax.experimental.pallas import tpu as pltpu
from jax.experimental.pallas import tpu_sc as plsc

N = 10000
E = 320000
T = 640000
A = 192
EMB = 128
TRIP = 64
RBF = 128
SPH = 7
NB = 3
CUT = 6.0

NPAD = 10240

EB = 1000
TB = 2000
NBLK = 1000


def _silu(x):
    return x * jax.lax.logistic(x)


def _node_init_kern(z_ref, lat_ref, tab_ref, wl1_ref, wl2_ref, h_ref):
    z = z_ref[...]
    ids = jax.lax.broadcasted_iota(jnp.int32, (z.shape[0], 128), 1)
    onehot = (z == ids).astype(jnp.float32)
    th = onehot @ tab_ref[...]
    x = jnp.concatenate([th, lat_ref[...]], axis=1)
    h_ref[...] = jax.nn.relu(x @ wl1_ref[...]) @ wl2_ref[...]


def _node_init(z, latent, atom_table, Wl1, Wl2):
    z2 = z.astype(jnp.int32).reshape(N, 1)
    tab = jnp.zeros((128, 128), jnp.float32).at[:100, :].set(atom_table)
    grid = N // NBLK
    return pl.pallas_call(
        _node_init_kern,
        grid=(grid,),
        in_specs=[
            pl.BlockSpec((NBLK, 1), lambda i: (i, 0)),
            pl.BlockSpec((NBLK, 64), lambda i: (i, 0)),
            pl.BlockSpec((128, 128), lambda i: (0, 0)),
            pl.BlockSpec((192, 192), lambda i: (0, 0)),
            pl.BlockSpec((192, 192), lambda i: (0, 0)),
        ],
        out_specs=pl.BlockSpec((NBLK, 192), lambda i: (i, 0)),
        out_shape=jax.ShapeDtypeStruct((N, 192), jnp.float32),
    )(z2, latent, tab, Wl1, Wl2)


def _edge_kern(ps_ref, pt_ref, hs_ref, ht_ref, we_ref, wr_ref,
               m_ref, unit_ref, rbfp_ref):
    vec = pt_ref[...] - ps_ref[...]
    d2 = jnp.sum(vec * vec, axis=1) + 1e-8
    D = jnp.sqrt(d2)
    unit = vec / D[:, None]
    dsc = D / CUT
    d5 = dsc * dsc * dsc * dsc * dsc
    env = jnp.where(
        dsc < 1.0,
        1.0 - 21.0 * d5 + 35.0 * d5 * dsc - 15.0 * d5 * dsc * dsc,
        0.0,
    )
    B = vec.shape[0]
    centers = (jax.lax.broadcasted_iota(jnp.int32, (B, RBF), 1)
               .astype(jnp.float32) * (CUT / (RBF - 1)))
    gamma = (RBF / CUT) ** 2
    rbf = env[:, None] * jnp.exp(-gamma * (D[:, None] - centers) ** 2)
    x = jnp.concatenate([hs_ref[...], ht_ref[...], rbf], axis=1)
    m_ref[...] = _silu(x @ we_ref[...])
    unit_ref[...] = unit
    rbfp_ref[...] = rbf @ wr_ref[...]


def _edge_stage(pos_s, pos_t, h_s, h_t, W_edge, W_rbf_all):
    grid = E // EB
    return pl.pallas_call(
        _edge_kern,
        grid=(grid,),
        in_specs=[
            pl.BlockSpec((EB, 16), lambda i: (i, 0)),
            pl.BlockSpec((EB, 16), lambda i: (i, 0)),
            pl.BlockSpec((EB, 192), lambda i: (i, 0)),
            pl.BlockSpec((EB, 192), lambda i: (i, 0)),
            pl.BlockSpec((512, 128), lambda i: (0, 0)),
            pl.BlockSpec((128, 64), lambda i: (0, 0)),
        ],
        out_specs=[
            pl.BlockSpec((EB, 128), lambda i: (i, 0)),
            pl.BlockSpec((EB, 16), lambda i: (i, 0)),
            pl.BlockSpec((EB, 64), lambda i: (i, 0)),
        ],
        out_shape=[
            jax.ShapeDtypeStruct((E, 128), jnp.float32),
            jax.ShapeDtypeStruct((E, 16), jnp.float32),
            jax.ShapeDtypeStruct((E, 64), jnp.float32),
        ],
    )(pos_s, pos_t, h_s, h_t, W_edge, W_rbf_all)


def _cbf_kern(uba_ref, uca_ref, rca_ref, wsph_ref, wup_ref, cbf_ref):
    cos = jnp.clip(jnp.sum(uba_ref[...] * uca_ref[...], axis=1), -1.0, 1.0)
    wsph = wsph_ref[...]
    C = cos.shape[0]
    acc = jnp.zeros((C, 16), jnp.float32)
    cp = jnp.ones((C,), jnp.float32)
    for l in range(SPH):
        acc = acc + cp[:, None] * wsph[l][None, :]
        cp = cp * cos
    cbf_ref[...] = (rca_ref[...] * acc) @ wup_ref[...]


def _cbf_stage(unit_ba, unit_ca, rbfc_ca, W_sph, W_cbf_up):
    wsph = jnp.zeros((8, 16), jnp.float32).at[:SPH, :].set(W_sph)
    grid = T // TB
    return pl.pallas_call(
        _cbf_kern,
        grid=(grid,),
        in_specs=[
            pl.BlockSpec((TB, 16), lambda i: (i, 0)),
            pl.BlockSpec((TB, 16), lambda i: (i, 0)),
            pl.BlockSpec((TB, 16), lambda i: (i, 0)),
            pl.BlockSpec((8, 16), lambda i: (0, 0)),
            pl.BlockSpec((16, 64), lambda i: (0, 0)),
        ],
        out_specs=pl.BlockSpec((TB, 64), lambda i: (i, 0)),
        out_shape=jax.ShapeDtypeStruct((T, 64), jnp.float32),
    )(unit_ba, unit_ca, rbfc_ca, wsph, W_cbf_up)


def _down_kern(m_ref, w1_ref, wd_ref, xd_ref):
    x = _silu(m_ref[...] @ w1_ref[...])
    xd_ref[...] = _silu(x @ wd_ref[...])


def _down_stage(m, W1, Wdown):
    grid = E // EB
    return pl.pallas_call(
        _down_kern,
        grid=(grid,),
        in_specs=[
            pl.BlockSpec((EB, 128), lambda i: (i, 0)),
            pl.BlockSpec((128, 128), lambda i: (0, 0)),
            pl.BlockSpec((128, 64), lambda i: (0, 0)),
        ],
        out_specs=pl.BlockSpec((EB, 64), lambda i: (i, 0)),
        out_shape=jax.ShapeDtypeStruct((E, 64), jnp.float32),
    )(m, W1, Wdown)


def _x3_kern(xd_ref, cbf_ref, x3_ref):
    x3_ref[...] = xd_ref[...] * cbf_ref[...]


def _x3_stage(xd_ba, cbf):
    grid = T // TB
    return pl.pallas_call(
        _x3_kern,
        grid=(grid,),
        in_specs=[
            pl.BlockSpec((TB, 64), lambda i: (i, 0)),
            pl.BlockSpec((TB, 64), lambda i: (i, 0)),
        ],
        out_specs=pl.BlockSpec((TB, 64), lambda i: (i, 0)),
        out_shape=jax.ShapeDtypeStruct((T, 64), jnp.float32),
    )(xd_ba, cbf)


def _mup_kern(m_ref, agg_ref, rbfp_ref, wup_ref, wg_ref, wh_ref,
              mn_ref, am_ref):
    rbf3g = rbfp_ref[...][:, 0:16]
    rbfhg = rbfp_ref[...][:, 16:32]
    m = m_ref[...] + _silu(agg_ref[...] @ wup_ref[...]) * (rbf3g @ wg_ref[...])
    mn_ref[...] = m
    am_ref[...] = m * (rbfhg @ wh_ref[...])


def _mup_stage(m, agg, rbfp, Wup, Wg, Wh):
    grid = E // EB
    return pl.pallas_call(
        _mup_kern,
        grid=(grid,),
        in_specs=[
            pl.BlockSpec((EB, 128), lambda i: (i, 0)),
            pl.BlockSpec((EB, 64), lambda i: (i, 0)),
            pl.BlockSpec((EB, 64), lambda i: (i, 0)),
            pl.BlockSpec((64, 128), lambda i: (0, 0)),
            pl.BlockSpec((16, 128), lambda i: (0, 0)),
            pl.BlockSpec((16, 128), lambda i: (0, 0)),
        ],
        out_specs=[
            pl.BlockSpec((EB, 128), lambda i: (i, 0)),
            pl.BlockSpec((EB, 128), lambda i: (i, 0)),
        ],
        out_shape=[
            jax.ShapeDtypeStruct((E, 128), jnp.float32),
            jax.ShapeDtypeStruct((E, 128), jnp.float32),
        ],
    )(m, agg, rbfp, Wup, Wg, Wh)


def _hup_kern(h_ref, ha0_ref, ha1_ref, was_ref, hn_ref):
    ha = ha0_ref[0] + ha1_ref[0]
    hn_ref[...] = h_ref[...] + _silu(ha @ was_ref[...])


def _hup_stage(h, ha2, Was):
    grid = N // NBLK
    return pl.pallas_call(
        _hup_kern,
        grid=(grid,),
        in_specs=[
            pl.BlockSpec((NBLK, 192), lambda i: (i, 0)),
            pl.BlockSpec((1, NBLK, 128), lambda i: (0, i, 0)),
            pl.BlockSpec((1, NBLK, 128), lambda i: (1, i, 0)),
            pl.BlockSpec((128, 192), lambda i: (0, 0)),
        ],
        out_specs=pl.BlockSpec((NBLK, 192), lambda i: (i, 0)),
        out_shape=jax.ShapeDtypeStruct((N, 192), jnp.float32),
    )(h, ha2, ha2, Was)


def _eres_kern(m_ref, hs_ref, ht_ref, wes_ref, mn_ref):
    x = hs_ref[...] + ht_ref[...]
    mn_ref[...] = m_ref[...] + _silu(x @ wes_ref[...])


def _eres_stage(m, h_s, h_t, Wes):
    grid = E // EB
    return pl.pallas_call(
        _eres_kern,
        grid=(grid,),
        in_specs=[
            pl.BlockSpec((EB, 128), lambda i: (i, 0)),
            pl.BlockSpec((EB, 192), lambda i: (i, 0)),
            pl.BlockSpec((EB, 192), lambda i: (i, 0)),
            pl.BlockSpec((192, 128), lambda i: (0, 0)),
        ],
        out_specs=pl.BlockSpec((EB, 128), lambda i: (i, 0)),
        out_shape=jax.ShapeDtypeStruct((E, 128), jnp.float32),
    )(m, h_s, h_t, Wes)


def _fout_kern(m_ref, rbfp_ref, unit_ref, wf_ref, fu_ref):
    gate = rbfp_ref[...][:, 32:48] @ wf_ref[...]
    f = jnp.sum(m_ref[...] * gate, axis=1)
    fu_ref[...] = f[:, None] * unit_ref[...]


def _fout_stage(m, rbfp, unit, Wout_f):
    grid = E // EB
    return pl.pallas_call(
        _fout_kern,
        grid=(grid,),
        in_specs=[
            pl.BlockSpec((EB, 128), lambda i: (i, 0)),
            pl.BlockSpec((EB, 64), lambda i: (i, 0)),
            pl.BlockSpec((EB, 16), lambda i: (i, 0)),
            pl.BlockSpec((16, 128), lambda i: (0, 0)),
        ],
        out_specs=pl.BlockSpec((EB, 16), lambda i: (i, 0)),
        out_shape=jax.ShapeDtypeStruct((E, 16), jnp.float32),
    )(m, rbfp, unit, Wout_f)


def _nout_kern(h_ref, f0_ref, f1_ref, we_ref, ea_ref, f_ref):
    ea_ref[...] = h_ref[...] @ we_ref[...]
    f_ref[...] = f0_ref[0] + f1_ref[0]


def _nout_stage(h, F2, Wout_e):
    wep = jnp.zeros((192, 128), jnp.float32).at[:, 0:1].set(Wout_e)
    grid = N // NBLK
    return pl.pallas_call(
        _nout_kern,
        grid=(grid,),
        in_specs=[
            pl.BlockSpec((NBLK, 192), lambda i: (i, 0)),
            pl.BlockSpec((1, NBLK, 16), lambda i: (0, i, 0)),
            pl.BlockSpec((1, NBLK, 16), lambda i: (1, i, 0)),
            pl.BlockSpec((192, 128), lambda i: (0, 0)),
        ],
        out_specs=[
            pl.BlockSpec((NBLK, 128), lambda i: (i, 0)),
            pl.BlockSpec((NBLK, 16), lambda i: (i, 0)),
        ],
        out_shape=[
            jax.ShapeDtypeStruct((N, 128), jnp.float32),
            jax.ShapeDtypeStruct((N, 16), jnp.float32),
        ],
    )(h, F2, F2, wep)


_NW = 32


@functools.cache
def _make_sc_gather(R, D, B, chunk):
    b_per_w = B // _NW
    iters = b_per_w // chunk
    assert b_per_w % chunk == 0 and chunk % 8 == 0
    mesh = plsc.VectorSubcoreMesh(core_axis_name="c", subcore_axis_name="s")

    @functools.partial(
        pl.kernel, mesh=mesh,
        out_type=jax.ShapeDtypeStruct((B, D), jnp.float32),
        compiler_params=pltpu.CompilerParams(use_tc_tiling_on_sc=False),
        scratch_types=[
            pltpu.VMEM((chunk,), jnp.int32),
            pltpu.VMEM((chunk, D), jnp.float32),
            pltpu.SemaphoreType.DMA,
        ],
    )
    def k(table_hbm, idx_hbm, out_hbm, idx_v, rows_v, sem):
        wid = lax.axis_index("s") * 2 + lax.axis_index("c")
        base = wid * b_per_w

        def body(j, carry):
            off = base + j * chunk
            pltpu.sync_copy(idx_hbm.at[pl.ds(off, chunk)], idx_v)
            pltpu.async_copy(table_hbm.at[idx_v], rows_v, sem).wait()
            pltpu.sync_copy(rows_v, out_hbm.at[pl.ds(off, chunk)])
            return carry

        lax.fori_loop(0, iters, body, 0)

    return k


def _gather(table, idx):
    R, D = table.shape
    B = idx.shape[0]
    b_per_w = B // _NW
    budget = min(2000, (400000 // (D * 4 + 4)) // 8 * 8)
    chunk = 8
    for c in range(budget, 7, -8):
        if b_per_w % c == 0 and c % 8 == 0:
            chunk = c
            break
    return _make_sc_gather(R, D, B, chunk)(table, idx)


def _segsum_n(rows, idx):
    s = jax.ops.segment_sum(rows, idx, num_segments=NPAD)
    return jnp.stack([s, jnp.zeros_like(s)], axis=0)


def _segsum_e(x3, id3_ca):
    return jax.ops.segment_sum(x3, id3_ca, num_segments=E)


def kernel(pos, latent, atom_table, Wl1, Wl2, W_edge, W_rbf3, W_rbf_h,
           W_rbf_out, W_rbf_cbf, W_sph, W_cbf_up, W1s, Wdowns, Wups, Wgs,
           Whs, Was, Wes, Wout_e, Wout_f, z, edge_index, id3_ba, id3_ca):
    idx_s = edge_index[0].astype(jnp.int32)
    idx_t = edge_index[1].astype(jnp.int32)
    ba = id3_ba.astype(jnp.int32)
    ca = id3_ca.astype(jnp.int32)

    pos16 = jnp.zeros((N, 16), jnp.float32).at[:, :3].set(pos)
    W_rbf_all = jnp.concatenate([W_rbf3, W_rbf_h, W_rbf_out, W_rbf_cbf], axis=1)

    h = _node_init(z, latent, atom_table, Wl1, Wl2)

    pos_s = _gather(pos16, idx_s)
    pos_t = _gather(pos16, idx_t)
    h_s = _gather(h, idx_s)
    h_t = _gather(h, idx_t)
    m, unit, rbfp = _edge_stage(pos_s, pos_t, h_s, h_t, W_edge, W_rbf_all)

    rbfc = rbfp[:, 48:64] + 0.0
    unit_ba = _gather(unit, ba)
    unit_ca = _gather(unit, ca)
    rbfc_ca = _gather(rbfc, ca)
    cbf = _cbf_stage(unit_ba, unit_ca, rbfc_ca, W_sph, W_cbf_up)

    for b in range(NB):
        xd = _down_stage(m, W1s[b], Wdowns[b])
        xd_ba = _gather(xd, ba)
        x3 = _x3_stage(xd_ba, cbf)
        agg = _segsum_e(x3, ca)
        m, am = _mup_stage(m, agg, rbfp, Wups[b], Wgs[b], Whs[b])
        ha2 = _segsum_n(am, idx_t)
        h = _hup_stage(h, ha2, Was[b])
        h_s = _gather(h, idx_s)
        h_t = _gather(h, idx_t)
        m = _eres_stage(m, h_s, h_t, Wes[b])

    fu = _fout_stage(m, rbfp, unit, Wout_f)
    F2 = _segsum_n(fu, idx_t)
    ea, F = _nout_stage(h, F2, Wout_e)
    return jnp.concatenate([ea[:, 0:1], F[:, 0:3]], axis=1)


# speedup vs baseline: 6.4388x; 1.1507x over previous
import functools
import jax
import jax.numpy as jnp
from jax import lax
from jax.experimental import pallas as pl
from jax.experimental.pallas import tpu as pltpu
from jax.experimental.pallas import tpu_sc as plsc

N = 10000
E = 320000
T = 640000
A = 192
EMB = 128
TRIP = 64
RBF = 128
SPH = 7
NB = 3
CUT = 6.0

NPAD = 10240

EB = 1000
TB = 2000
NBLK = 1000


def _silu(x):
    return x * jax.lax.logistic(x)


def _node_init_kern(z_ref, lat_ref, tab_ref, wl1_ref, wl2_ref, h_ref):
    z = z_ref[...]
    ids = jax.lax.broadcasted_iota(jnp.int32, (z.shape[0], 128), 1)
    onehot = (z == ids).astype(jnp.float32)
    th = onehot @ tab_ref[...]
    x = jnp.concatenate([th, lat_ref[...]], axis=1)
    h_ref[...] = jax.nn.relu(x @ wl1_ref[...]) @ wl2_ref[...]


def _node_init(z, latent, atom_table, Wl1, Wl2):
    z2 = z.astype(jnp.int32).reshape(N, 1)
    tab = jnp.zeros((128, 128), jnp.float32).at[:100, :].set(atom_table)
    grid = N // NBLK
    return pl.pallas_call(
        _node_init_kern,
        grid=(grid,),
        in_specs=[
            pl.BlockSpec((NBLK, 1), lambda i: (i, 0)),
            pl.BlockSpec((NBLK, 64), lambda i: (i, 0)),
            pl.BlockSpec((128, 128), lambda i: (0, 0)),
            pl.BlockSpec((192, 192), lambda i: (0, 0)),
            pl.BlockSpec((192, 192), lambda i: (0, 0)),
        ],
        out_specs=pl.BlockSpec((NBLK, 192), lambda i: (i, 0)),
        out_shape=jax.ShapeDtypeStruct((N, 192), jnp.float32),
    )(z2, latent, tab, Wl1, Wl2)


def _edge_kern(ps_ref, pt_ref, hs_ref, ht_ref, we_ref, wr_ref,
               m_ref, unit_ref, rbfp_ref):
    vec = pt_ref[...] - ps_ref[...]
    d2 = jnp.sum(vec * vec, axis=1) + 1e-8
    D = jnp.sqrt(d2)
    unit = vec / D[:, None]
    dsc = D / CUT
    d5 = dsc * dsc * dsc * dsc * dsc
    env = jnp.where(
        dsc < 1.0,
        1.0 - 21.0 * d5 + 35.0 * d5 * dsc - 15.0 * d5 * dsc * dsc,
        0.0,
    )
    B = vec.shape[0]
    centers = (jax.lax.broadcasted_iota(jnp.int32, (B, RBF), 1)
               .astype(jnp.float32) * (CUT / (RBF - 1)))
    gamma = (RBF / CUT) ** 2
    rbf = env[:, None] * jnp.exp(-gamma * (D[:, None] - centers) ** 2)
    x = jnp.concatenate([hs_ref[...], ht_ref[...], rbf], axis=1)
    m_ref[...] = _silu(x @ we_ref[...])
    unit_ref[...] = unit
    rbfp_ref[...] = rbf @ wr_ref[...]


def _edge_stage(pos_s, pos_t, h_s, h_t, W_edge, W_rbf_all):
    grid = E // EB
    return pl.pallas_call(
        _edge_kern,
        grid=(grid,),
        in_specs=[
            pl.BlockSpec((EB, 16), lambda i: (i, 0)),
            pl.BlockSpec((EB, 16), lambda i: (i, 0)),
            pl.BlockSpec((EB, 192), lambda i: (i, 0)),
            pl.BlockSpec((EB, 192), lambda i: (i, 0)),
            pl.BlockSpec((512, 128), lambda i: (0, 0)),
            pl.BlockSpec((128, 64), lambda i: (0, 0)),
        ],
        out_specs=[
            pl.BlockSpec((EB, 128), lambda i: (i, 0)),
            pl.BlockSpec((EB, 16), lambda i: (i, 0)),
            pl.BlockSpec((EB, 64), lambda i: (i, 0)),
        ],
        out_shape=[
            jax.ShapeDtypeStruct((E, 128), jnp.float32),
            jax.ShapeDtypeStruct((E, 16), jnp.float32),
            jax.ShapeDtypeStruct((E, 64), jnp.float32),
        ],
    )(pos_s, pos_t, h_s, h_t, W_edge, W_rbf_all)


def _cbf_kern(uba_ref, uca_ref, rca_ref, wsph_ref, wup_ref, cbf_ref):
    cos = jnp.clip(jnp.sum(uba_ref[...] * uca_ref[...], axis=1), -1.0, 1.0)
    wsph = wsph_ref[...]
    C = cos.shape[0]
    acc = jnp.zeros((C, 16), jnp.float32)
    cp = jnp.ones((C,), jnp.float32)
    for l in range(SPH):
        acc = acc + cp[:, None] * wsph[l][None, :]
        cp = cp * cos
    cbf_ref[...] = (rca_ref[...] * acc) @ wup_ref[...]


def _cbf_stage(unit_ba, unit_ca, rbfc_ca, W_sph, W_cbf_up):
    wsph = jnp.zeros((8, 16), jnp.float32).at[:SPH, :].set(W_sph)
    grid = T // TB
    return pl.pallas_call(
        _cbf_kern,
        grid=(grid,),
        in_specs=[
            pl.BlockSpec((TB, 16), lambda i: (i, 0)),
            pl.BlockSpec((TB, 16), lambda i: (i, 0)),
            pl.BlockSpec((TB, 16), lambda i: (i, 0)),
            pl.BlockSpec((8, 16), lambda i: (0, 0)),
            pl.BlockSpec((16, 64), lambda i: (0, 0)),
        ],
        out_specs=pl.BlockSpec((TB, 64), lambda i: (i, 0)),
        out_shape=jax.ShapeDtypeStruct((T, 64), jnp.float32),
    )(unit_ba, unit_ca, rbfc_ca, wsph, W_cbf_up)


def _down_kern(m_ref, w1_ref, wd_ref, xd_ref):
    x = _silu(m_ref[...] @ w1_ref[...])
    xd_ref[...] = _silu(x @ wd_ref[...])


def _down_stage(m, W1, Wdown):
    grid = E // EB
    return pl.pallas_call(
        _down_kern,
        grid=(grid,),
        in_specs=[
            pl.BlockSpec((EB, 128), lambda i: (i, 0)),
            pl.BlockSpec((128, 128), lambda i: (0, 0)),
            pl.BlockSpec((128, 64), lambda i: (0, 0)),
        ],
        out_specs=pl.BlockSpec((EB, 64), lambda i: (i, 0)),
        out_shape=jax.ShapeDtypeStruct((E, 64), jnp.float32),
    )(m, W1, Wdown)


def _x3_kern(xd_ref, cbf_ref, x3_ref):
    x3_ref[...] = xd_ref[...] * cbf_ref[...]


def _x3_stage(xd_ba, cbf):
    grid = T // TB
    return pl.pallas_call(
        _x3_kern,
        grid=(grid,),
        in_specs=[
            pl.BlockSpec((TB, 64), lambda i: (i, 0)),
            pl.BlockSpec((TB, 64), lambda i: (i, 0)),
        ],
        out_specs=pl.BlockSpec((TB, 64), lambda i: (i, 0)),
        out_shape=jax.ShapeDtypeStruct((T, 64), jnp.float32),
    )(xd_ba, cbf)


def _mup_kern(m_ref, agg_ref, rbfp_ref, wup_ref, wg_ref, wh_ref,
              mn_ref, am_ref):
    rbf3g = rbfp_ref[...][:, 0:16]
    rbfhg = rbfp_ref[...][:, 16:32]
    m = m_ref[...] + _silu(agg_ref[...] @ wup_ref[...]) * (rbf3g @ wg_ref[...])
    mn_ref[...] = m
    am_ref[...] = m * (rbfhg @ wh_ref[...])


def _mup_stage(m, agg, rbfp, Wup, Wg, Wh):
    grid = E // EB
    return pl.pallas_call(
        _mup_kern,
        grid=(grid,),
        in_specs=[
            pl.BlockSpec((EB, 128), lambda i: (i, 0)),
            pl.BlockSpec((EB, 64), lambda i: (i, 0)),
            pl.BlockSpec((EB, 64), lambda i: (i, 0)),
            pl.BlockSpec((64, 128), lambda i: (0, 0)),
            pl.BlockSpec((16, 128), lambda i: (0, 0)),
            pl.BlockSpec((16, 128), lambda i: (0, 0)),
        ],
        out_specs=[
            pl.BlockSpec((EB, 128), lambda i: (i, 0)),
            pl.BlockSpec((EB, 128), lambda i: (i, 0)),
        ],
        out_shape=[
            jax.ShapeDtypeStruct((E, 128), jnp.float32),
            jax.ShapeDtypeStruct((E, 128), jnp.float32),
        ],
    )(m, agg, rbfp, Wup, Wg, Wh)


def _hup_kern(h_ref, ha0_ref, ha1_ref, was_ref, hn_ref):
    ha = ha0_ref[0] + ha1_ref[0]
    hn_ref[...] = h_ref[...] + _silu(ha @ was_ref[...])


def _hup_stage(h, ha2, Was):
    grid = N // NBLK
    return pl.pallas_call(
        _hup_kern,
        grid=(grid,),
        in_specs=[
            pl.BlockSpec((NBLK, 192), lambda i: (i, 0)),
            pl.BlockSpec((1, NBLK, 128), lambda i: (0, i, 0)),
            pl.BlockSpec((1, NBLK, 128), lambda i: (1, i, 0)),
            pl.BlockSpec((128, 192), lambda i: (0, 0)),
        ],
        out_specs=pl.BlockSpec((NBLK, 192), lambda i: (i, 0)),
        out_shape=jax.ShapeDtypeStruct((N, 192), jnp.float32),
    )(h, ha2, ha2, Was)


def _eres_kern(m_ref, hs_ref, ht_ref, wes_ref, mn_ref):
    x = hs_ref[...] + ht_ref[...]
    mn_ref[...] = m_ref[...] + _silu(x @ wes_ref[...])


def _eres_stage(m, h_s, h_t, Wes):
    grid = E // EB
    return pl.pallas_call(
        _eres_kern,
        grid=(grid,),
        in_specs=[
            pl.BlockSpec((EB, 128), lambda i: (i, 0)),
            pl.BlockSpec((EB, 192), lambda i: (i, 0)),
            pl.BlockSpec((EB, 192), lambda i: (i, 0)),
            pl.BlockSpec((192, 128), lambda i: (0, 0)),
        ],
        out_specs=pl.BlockSpec((EB, 128), lambda i: (i, 0)),
        out_shape=jax.ShapeDtypeStruct((E, 128), jnp.float32),
    )(m, h_s, h_t, Wes)


def _fout_kern(m_ref, rbfp_ref, unit_ref, wf_ref, fu_ref):
    gate = rbfp_ref[...][:, 32:48] @ wf_ref[...]
    f = jnp.sum(m_ref[...] * gate, axis=1)
    fu_ref[...] = f[:, None] * unit_ref[...]


def _fout_stage(m, rbfp, unit, Wout_f):
    grid = E // EB
    return pl.pallas_call(
        _fout_kern,
        grid=(grid,),
        in_specs=[
            pl.BlockSpec((EB, 128), lambda i: (i, 0)),
            pl.BlockSpec((EB, 64), lambda i: (i, 0)),
            pl.BlockSpec((EB, 16), lambda i: (i, 0)),
            pl.BlockSpec((16, 128), lambda i: (0, 0)),
        ],
        out_specs=pl.BlockSpec((EB, 16), lambda i: (i, 0)),
        out_shape=jax.ShapeDtypeStruct((E, 16), jnp.float32),
    )(m, rbfp, unit, Wout_f)


def _nout_kern(h_ref, f0_ref, f1_ref, we_ref, ea_ref, f_ref):
    ea_ref[...] = h_ref[...] @ we_ref[...]
    f_ref[...] = f0_ref[0] + f1_ref[0]


def _nout_stage(h, F2, Wout_e):
    wep = jnp.zeros((192, 128), jnp.float32).at[:, 0:1].set(Wout_e)
    grid = N // NBLK
    return pl.pallas_call(
        _nout_kern,
        grid=(grid,),
        in_specs=[
            pl.BlockSpec((NBLK, 192), lambda i: (i, 0)),
            pl.BlockSpec((1, NBLK, 16), lambda i: (0, i, 0)),
            pl.BlockSpec((1, NBLK, 16), lambda i: (1, i, 0)),
            pl.BlockSpec((192, 128), lambda i: (0, 0)),
        ],
        out_specs=[
            pl.BlockSpec((NBLK, 128), lambda i: (i, 0)),
            pl.BlockSpec((NBLK, 16), lambda i: (i, 0)),
        ],
        out_shape=[
            jax.ShapeDtypeStruct((N, 128), jnp.float32),
            jax.ShapeDtypeStruct((N, 16), jnp.float32),
        ],
    )(h, F2, F2, wep)


_NW = 32


@functools.cache
def _make_sc_gather(R, D, B, chunk):
    b_per_w = B // _NW
    iters = b_per_w // chunk
    assert b_per_w % chunk == 0 and chunk % 8 == 0
    mesh = plsc.VectorSubcoreMesh(core_axis_name="c", subcore_axis_name="s")

    @functools.partial(
        pl.kernel, mesh=mesh,
        out_type=jax.ShapeDtypeStruct((B, D), jnp.float32),
        compiler_params=pltpu.CompilerParams(use_tc_tiling_on_sc=False),
        scratch_types=[
            pltpu.VMEM((chunk,), jnp.int32),
            pltpu.VMEM((chunk, D), jnp.float32),
            pltpu.SemaphoreType.DMA,
        ],
    )
    def k(table_hbm, idx_hbm, out_hbm, idx_v, rows_v, sem):
        wid = lax.axis_index("s") * 2 + lax.axis_index("c")
        base = wid * b_per_w

        def body(j, carry):
            off = base + j * chunk
            pltpu.sync_copy(idx_hbm.at[pl.ds(off, chunk)], idx_v)
            pltpu.async_copy(table_hbm.at[idx_v], rows_v, sem).wait()
            pltpu.sync_copy(rows_v, out_hbm.at[pl.ds(off, chunk)])
            return carry

        lax.fori_loop(0, iters, body, 0)

    return k


@functools.cache
def _make_sc_gather_mul(R, D, B, chunk):
    b_per_w = B // _NW
    iters = b_per_w // chunk
    assert b_per_w % chunk == 0 and chunk % 8 == 0 and D % 16 == 0
    mesh = plsc.VectorSubcoreMesh(core_axis_name="c", subcore_axis_name="s")

    @functools.partial(
        pl.kernel, mesh=mesh,
        out_type=jax.ShapeDtypeStruct((B, D), jnp.float32),
        compiler_params=pltpu.CompilerParams(use_tc_tiling_on_sc=False),
        scratch_types=[
            pltpu.VMEM((chunk,), jnp.int32),
            pltpu.VMEM((chunk, D), jnp.float32),
            pltpu.VMEM((chunk, D), jnp.float32),
            pltpu.SemaphoreType.DMA,
        ],
    )
    def k(table_hbm, idx_hbm, mul_hbm, out_hbm, idx_v, rows_v, mul_v, sem):
        wid = lax.axis_index("s") * 2 + lax.axis_index("c")
        base = wid * b_per_w
        nv = D // 16

        def body(j, carry):
            off = base + j * chunk
            pltpu.sync_copy(idx_hbm.at[pl.ds(off, chunk)], idx_v)
            pltpu.sync_copy(mul_hbm.at[pl.ds(off, chunk)], mul_v)
            pltpu.async_copy(table_hbm.at[idx_v], rows_v, sem).wait()

            def row(i, c2):
                for v in range(nv):
                    sl = pl.ds(v * 16, 16)
                    rows_v[i, sl] = rows_v[i, sl] * mul_v[i, sl]
                return c2

            lax.fori_loop(0, chunk, row, 0)
            pltpu.sync_copy(rows_v, out_hbm.at[pl.ds(off, chunk)])
            return carry

        lax.fori_loop(0, iters, body, 0)

    return k


@functools.cache
def _make_sc_scatter_add(S, D, B, chunk):
    b_per_c = B // 2
    b_per_w = B // _NW
    iters = b_per_w // chunk
    stripe = S // 16
    assert b_per_w % chunk == 0 and chunk % 8 == 0 and S % 16 == 0
    assert stripe % 8 == 0
    mesh = plsc.VectorSubcoreMesh(core_axis_name="c", subcore_axis_name="s")

    @functools.partial(
        pl.kernel, mesh=mesh,
        out_type=jax.ShapeDtypeStruct((2, S, D), jnp.float32),
        compiler_params=pltpu.CompilerParams(use_tc_tiling_on_sc=False),
        scratch_types=[
            pltpu.VMEM_SHARED((S, D), jnp.float32),
            pltpu.VMEM((chunk,), jnp.int32),
            pltpu.VMEM((chunk, D), jnp.float32),
        ],
    )
    def k(rows_hbm, idx_hbm, zeros_hbm, out_hbm, acc_sh, idx_v, rows_v):
        c = lax.axis_index("c")
        s = lax.axis_index("s")
        pltpu.sync_copy(zeros_hbm, acc_sh.at[pl.ds(s * stripe, stripe)])
        plsc.subcore_barrier()
        base = c * b_per_c + s * b_per_w

        def body(j, carry):
            off = base + j * chunk
            pltpu.sync_copy(idx_hbm.at[pl.ds(off, chunk)], idx_v)
            pltpu.sync_copy(rows_hbm.at[pl.ds(off, chunk)], rows_v)
            pltpu.sync_copy(rows_v, acc_sh.at[idx_v], add=True)
            return carry

        lax.fori_loop(0, iters, body, 0)
        plsc.subcore_barrier()
        pltpu.sync_copy(acc_sh.at[pl.ds(s * stripe, stripe)],
                        out_hbm.at[c, pl.ds(s * stripe, stripe)])

    return k


def _gather(table, idx):
    R, D = table.shape
    B = idx.shape[0]
    b_per_w = B // _NW
    budget = min(2000, (400000 // (D * 4 + 4)) // 8 * 8)
    chunk = 8
    for c in range(budget, 7, -8):
        if b_per_w % c == 0 and c % 8 == 0:
            chunk = c
            break
    return _make_sc_gather(R, D, B, chunk)(table, idx)


def _gather_mul(table, idx, mul):
    R, D = table.shape
    B = idx.shape[0]
    b_per_w = B // _NW
    budget = (400000 // (D * 8 + 4)) // 8 * 8
    chunk = 8
    for c in range(budget, 7, -8):
        if b_per_w % c == 0:
            chunk = c
            break
    return _make_sc_gather_mul(R, D, B, chunk)(table, idx, mul)


def _segsum_n(rows, idx):
    B, D = rows.shape
    b_per_w = B // _NW
    budget = (400000 // (D * 4 + 4)) // 8 * 8
    spare = (1900000 - NPAD * D) // (16 * D) // 8 * 8
    budget = min(budget, spare)
    chunk = 8
    for c in range(budget, 7, -8):
        if b_per_w % c == 0:
            chunk = c
            break
    zeros = jnp.zeros((NPAD // 16, D), jnp.float32)
    return _make_sc_scatter_add(NPAD, D, B, chunk)(rows, idx, zeros)


def _segsum_e(x3, id3_ca):
    return jax.ops.segment_sum(x3, id3_ca, num_segments=E)


def kernel(pos, latent, atom_table, Wl1, Wl2, W_edge, W_rbf3, W_rbf_h,
           W_rbf_out, W_rbf_cbf, W_sph, W_cbf_up, W1s, Wdowns, Wups, Wgs,
           Whs, Was, Wes, Wout_e, Wout_f, z, edge_index, id3_ba, id3_ca):
    idx_s = edge_index[0].astype(jnp.int32)
    idx_t = edge_index[1].astype(jnp.int32)
    ba = id3_ba.astype(jnp.int32)
    ca = id3_ca.astype(jnp.int32)

    pos16 = jnp.zeros((N, 16), jnp.float32).at[:, :3].set(pos)
    W_rbf_all = jnp.concatenate([W_rbf3, W_rbf_h, W_rbf_out, W_rbf_cbf], axis=1)

    h = _node_init(z, latent, atom_table, Wl1, Wl2)

    pos_s = _gather(pos16, idx_s)
    pos_t = _gather(pos16, idx_t)
    h_s = _gather(h, idx_s)
    h_t = _gather(h, idx_t)
    m, unit, rbfp = _edge_stage(pos_s, pos_t, h_s, h_t, W_edge, W_rbf_all)

    rbfc = rbfp[:, 48:64] + 0.0
    unit_ba = _gather(unit, ba)
    unit_ca = _gather(unit, ca)
    rbfc_ca = _gather(rbfc, ca)
    cbf = _cbf_stage(unit_ba, unit_ca, rbfc_ca, W_sph, W_cbf_up)

    for b in range(NB):
        xd = _down_stage(m, W1s[b], Wdowns[b])
        x3 = _gather_mul(xd, ba, cbf)
        agg = _segsum_e(x3, ca)
        m, am = _mup_stage(m, agg, rbfp, Wups[b], Wgs[b], Whs[b])
        ha2 = _segsum_n(am, idx_t)
        h = _hup_stage(h, ha2, Was[b])
        h_s = _gather(h, idx_s)
        h_t = _gather(h, idx_t)
        m = _eres_stage(m, h_s, h_t, Wes[b])

    fu = _fout_stage(m, rbfp, unit, Wout_f)
    F2 = _segsum_n(fu, idx_t)
    ea, F = _nout_stage(h, F2, Wout_e)
    return jnp.concatenate([ea[:, 0:1], F[:, 0:3]], axis=1)


# speedup vs baseline: 8.3269x; 1.2932x over previous
import functools
import jax
import jax.numpy as jnp
from jax import lax
from jax.experimental import pallas as pl
from jax.experimental.pallas import tpu as pltpu
from jax.experimental.pallas import tpu_sc as plsc

N = 10000
E = 320000
T = 640000
A = 192
EMB = 128
TRIP = 64
RBF = 128
SPH = 7
NB = 3
CUT = 6.0

NPAD = 10240

EB = 1000
TB = 2240
NBLK = 1000


def _silu(x):
    return x * jax.lax.logistic(x)


def _node_init_kern(z_ref, lat_ref, tab_ref, wl1_ref, wl2_ref, h_ref):
    z = z_ref[...]
    ids = jax.lax.broadcasted_iota(jnp.int32, (z.shape[0], 128), 1)
    onehot = (z == ids).astype(jnp.float32)
    th = onehot @ tab_ref[...]
    x = jnp.concatenate([th, lat_ref[...]], axis=1)
    h_ref[...] = jax.nn.relu(x @ wl1_ref[...]) @ wl2_ref[...]


def _node_init(z, latent, atom_table, Wl1, Wl2):
    z2 = z.astype(jnp.int32).reshape(N, 1)
    tab = jnp.zeros((128, 128), jnp.float32).at[:100, :].set(atom_table)
    grid = N // NBLK
    return pl.pallas_call(
        _node_init_kern,
        grid=(grid,),
        in_specs=[
            pl.BlockSpec((NBLK, 1), lambda i: (i, 0)),
            pl.BlockSpec((NBLK, 64), lambda i: (i, 0)),
            pl.BlockSpec((128, 128), lambda i: (0, 0)),
            pl.BlockSpec((192, 192), lambda i: (0, 0)),
            pl.BlockSpec((192, 192), lambda i: (0, 0)),
        ],
        out_specs=pl.BlockSpec((NBLK, 192), lambda i: (i, 0)),
        out_shape=jax.ShapeDtypeStruct((N, 192), jnp.float32),
    )(z2, latent, tab, Wl1, Wl2)


def _edge_kern(ps_ref, pt_ref, hs_ref, ht_ref, we_ref, wr_ref,
               m_ref, unit_ref, rbfp_ref):
    vec = pt_ref[...] - ps_ref[...]
    d2 = jnp.sum(vec * vec, axis=1) + 1e-8
    D = jnp.sqrt(d2)
    unit = vec / D[:, None]
    dsc = D / CUT
    d5 = dsc * dsc * dsc * dsc * dsc
    env = jnp.where(
        dsc < 1.0,
        1.0 - 21.0 * d5 + 35.0 * d5 * dsc - 15.0 * d5 * dsc * dsc,
        0.0,
    )
    B = vec.shape[0]
    centers = (jax.lax.broadcasted_iota(jnp.int32, (B, RBF), 1)
               .astype(jnp.float32) * (CUT / (RBF - 1)))
    gamma = (RBF / CUT) ** 2
    rbf = env[:, None] * jnp.exp(-gamma * (D[:, None] - centers) ** 2)
    x = jnp.concatenate([hs_ref[...], ht_ref[...], rbf], axis=1)
    m_ref[...] = _silu(x @ we_ref[...])
    unit_ref[...] = unit
    rbfp_ref[...] = rbf @ wr_ref[...]


def _edge_stage(pos_s, pos_t, h_s, h_t, W_edge, W_rbf_all):
    grid = E // EB
    return pl.pallas_call(
        _edge_kern,
        grid=(grid,),
        in_specs=[
            pl.BlockSpec((EB, 16), lambda i: (i, 0)),
            pl.BlockSpec((EB, 16), lambda i: (i, 0)),
            pl.BlockSpec((EB, 192), lambda i: (i, 0)),
            pl.BlockSpec((EB, 192), lambda i: (i, 0)),
            pl.BlockSpec((512, 128), lambda i: (0, 0)),
            pl.BlockSpec((128, 64), lambda i: (0, 0)),
        ],
        out_specs=[
            pl.BlockSpec((EB, 128), lambda i: (i, 0)),
            pl.BlockSpec((EB, 16), lambda i: (i, 0)),
            pl.BlockSpec((EB, 64), lambda i: (i, 0)),
        ],
        out_shape=[
            jax.ShapeDtypeStruct((E, 128), jnp.float32),
            jax.ShapeDtypeStruct((E, 16), jnp.float32),
            jax.ShapeDtypeStruct((E, 64), jnp.float32),
        ],
    )(pos_s, pos_t, h_s, h_t, W_edge, W_rbf_all)


def _cbf_kern(uba_ref, uca_ref, rca_ref, wsph_ref, wup_ref, cbf_ref):
    cos = jnp.clip(jnp.sum(uba_ref[...] * uca_ref[...], axis=1), -1.0, 1.0)
    wsph = wsph_ref[...]
    C = cos.shape[0]
    acc = jnp.zeros((C, 16), jnp.float32)
    cp = jnp.ones((C,), jnp.float32)
    for l in range(SPH):
        acc = acc + cp[:, None] * wsph[l][None, :]
        cp = cp * cos
    cbf_ref[...] = (rca_ref[...] * acc) @ wup_ref[...]


def _cbf_stage(unit_ba, unit_ca, rbfc_ca, W_sph, W_cbf_up):
    wsph = jnp.zeros((8, 16), jnp.float32).at[:SPH, :].set(W_sph)
    grid = TP // TB
    return pl.pallas_call(
        _cbf_kern,
        grid=(grid,),
        in_specs=[
            pl.BlockSpec((TB, 16), lambda i: (i, 0)),
            pl.BlockSpec((TB, 16), lambda i: (i, 0)),
            pl.BlockSpec((TB, 16), lambda i: (i, 0)),
            pl.BlockSpec((8, 16), lambda i: (0, 0)),
            pl.BlockSpec((16, 64), lambda i: (0, 0)),
        ],
        out_specs=pl.BlockSpec((TB, 64), lambda i: (i, 0)),
        out_shape=jax.ShapeDtypeStruct((TP, 64), jnp.float32),
    )(unit_ba, unit_ca, rbfc_ca, wsph, W_cbf_up)


def _down_kern(m_ref, w1_ref, wd_ref, xd_ref):
    x = _silu(m_ref[...] @ w1_ref[...])
    xd_ref[...] = _silu(x @ wd_ref[...])


def _down_stage(m, W1, Wdown):
    grid = E // EB
    return pl.pallas_call(
        _down_kern,
        grid=(grid,),
        in_specs=[
            pl.BlockSpec((EB, 128), lambda i: (i, 0)),
            pl.BlockSpec((128, 128), lambda i: (0, 0)),
            pl.BlockSpec((128, 64), lambda i: (0, 0)),
        ],
        out_specs=pl.BlockSpec((EB, 64), lambda i: (i, 0)),
        out_shape=jax.ShapeDtypeStruct((E, 64), jnp.float32),
    )(m, W1, Wdown)


def _mup_kern(m_ref, agg_ref, rbfp_ref, wup_ref, wg_ref, wh_ref,
              mn_ref, am_ref):
    rbf3g = rbfp_ref[...][:, 0:16]
    rbfhg = rbfp_ref[...][:, 16:32]
    m = m_ref[...] + _silu(agg_ref[...] @ wup_ref[...]) * (rbf3g @ wg_ref[...])
    mn_ref[...] = m
    am_ref[...] = m * (rbfhg @ wh_ref[...])


def _mup_stage(m, agg, rbfp, Wup, Wg, Wh):
    grid = E // EB
    return pl.pallas_call(
        _mup_kern,
        grid=(grid,),
        in_specs=[
            pl.BlockSpec((EB, 128), lambda i: (i, 0)),
            pl.BlockSpec((EB, 64), lambda i: (i, 0)),
            pl.BlockSpec((EB, 64), lambda i: (i, 0)),
            pl.BlockSpec((64, 128), lambda i: (0, 0)),
            pl.BlockSpec((16, 128), lambda i: (0, 0)),
            pl.BlockSpec((16, 128), lambda i: (0, 0)),
        ],
        out_specs=[
            pl.BlockSpec((EB, 128), lambda i: (i, 0)),
            pl.BlockSpec((EB, 128), lambda i: (i, 0)),
        ],
        out_shape=[
            jax.ShapeDtypeStruct((E, 128), jnp.float32),
            jax.ShapeDtypeStruct((E, 128), jnp.float32),
        ],
    )(m, agg, rbfp, Wup, Wg, Wh)


def _hup_kern(h_ref, ha0_ref, ha1_ref, was_ref, hn_ref):
    ha = ha0_ref[0] + ha1_ref[0]
    hn_ref[...] = h_ref[...] + _silu(ha @ was_ref[...])


def _hup_stage(h, ha2, Was):
    grid = N // NBLK
    return pl.pallas_call(
        _hup_kern,
        grid=(grid,),
        in_specs=[
            pl.BlockSpec((NBLK, 192), lambda i: (i, 0)),
            pl.BlockSpec((1, NBLK, 128), lambda i: (0, i, 0)),
            pl.BlockSpec((1, NBLK, 128), lambda i: (1, i, 0)),
            pl.BlockSpec((128, 192), lambda i: (0, 0)),
        ],
        out_specs=pl.BlockSpec((NBLK, 192), lambda i: (i, 0)),
        out_shape=jax.ShapeDtypeStruct((N, 192), jnp.float32),
    )(h, ha2, ha2, Was)


def _eres_kern(m_ref, hs_ref, ht_ref, wes_ref, mn_ref):
    x = hs_ref[...] + ht_ref[...]
    mn_ref[...] = m_ref[...] + _silu(x @ wes_ref[...])


def _eres_stage(m, h_s, h_t, Wes):
    grid = E // EB
    return pl.pallas_call(
        _eres_kern,
        grid=(grid,),
        in_specs=[
            pl.BlockSpec((EB, 128), lambda i: (i, 0)),
            pl.BlockSpec((EB, 192), lambda i: (i, 0)),
            pl.BlockSpec((EB, 192), lambda i: (i, 0)),
            pl.BlockSpec((192, 128), lambda i: (0, 0)),
        ],
        out_specs=pl.BlockSpec((EB, 128), lambda i: (i, 0)),
        out_shape=jax.ShapeDtypeStruct((E, 128), jnp.float32),
    )(m, h_s, h_t, Wes)


def _fout_kern(m_ref, rbfp_ref, unit_ref, wf_ref, fu_ref):
    gate = rbfp_ref[...][:, 32:48] @ wf_ref[...]
    f = jnp.sum(m_ref[...] * gate, axis=1)
    fu_ref[...] = f[:, None] * unit_ref[...]


def _fout_stage(m, rbfp, unit, Wout_f):
    grid = E // EB
    return pl.pallas_call(
        _fout_kern,
        grid=(grid,),
        in_specs=[
            pl.BlockSpec((EB, 128), lambda i: (i, 0)),
            pl.BlockSpec((EB, 64), lambda i: (i, 0)),
            pl.BlockSpec((EB, 16), lambda i: (i, 0)),
            pl.BlockSpec((16, 128), lambda i: (0, 0)),
        ],
        out_specs=pl.BlockSpec((EB, 16), lambda i: (i, 0)),
        out_shape=jax.ShapeDtypeStruct((E, 16), jnp.float32),
    )(m, rbfp, unit, Wout_f)


def _nout_kern(h_ref, f0_ref, f1_ref, we_ref, ea_ref, f_ref):
    ea_ref[...] = h_ref[...] @ we_ref[...]
    f_ref[...] = f0_ref[0] + f1_ref[0]


def _nout_stage(h, F2, Wout_e):
    wep = jnp.zeros((192, 128), jnp.float32).at[:, 0:1].set(Wout_e)
    grid = N // NBLK
    return pl.pallas_call(
        _nout_kern,
        grid=(grid,),
        in_specs=[
            pl.BlockSpec((NBLK, 192), lambda i: (i, 0)),
            pl.BlockSpec((1, NBLK, 16), lambda i: (0, i, 0)),
            pl.BlockSpec((1, NBLK, 16), lambda i: (1, i, 0)),
            pl.BlockSpec((192, 128), lambda i: (0, 0)),
        ],
        out_specs=[
            pl.BlockSpec((NBLK, 128), lambda i: (i, 0)),
            pl.BlockSpec((NBLK, 16), lambda i: (i, 0)),
        ],
        out_shape=[
            jax.ShapeDtypeStruct((N, 128), jnp.float32),
            jax.ShapeDtypeStruct((N, 16), jnp.float32),
        ],
    )(h, F2, F2, wep)


_NW = 32


@functools.cache
def _make_sc_gather(R, D, B, chunk):
    b_per_w = B // _NW
    iters = b_per_w // chunk
    assert b_per_w % chunk == 0 and chunk % 8 == 0
    mesh = plsc.VectorSubcoreMesh(core_axis_name="c", subcore_axis_name="s")

    @functools.partial(
        pl.kernel, mesh=mesh,
        out_type=jax.ShapeDtypeStruct((B, D), jnp.float32),
        compiler_params=pltpu.CompilerParams(use_tc_tiling_on_sc=False),
        scratch_types=[
            pltpu.VMEM((chunk,), jnp.int32),
            pltpu.VMEM((chunk, D), jnp.float32),
            pltpu.SemaphoreType.DMA,
        ],
    )
    def k(table_hbm, idx_hbm, out_hbm, idx_v, rows_v, sem):
        wid = lax.axis_index("s") * 2 + lax.axis_index("c")
        base = wid * b_per_w

        def body(j, carry):
            off = base + j * chunk
            pltpu.sync_copy(idx_hbm.at[pl.ds(off, chunk)], idx_v)
            pltpu.async_copy(table_hbm.at[idx_v], rows_v, sem).wait()
            pltpu.sync_copy(rows_v, out_hbm.at[pl.ds(off, chunk)])
            return carry

        lax.fori_loop(0, iters, body, 0)

    return k


TP = 645120
EBLK = 4000
NROUND = E // EBLK // 2
ACC = 4096
TCH = 512


@functools.cache
def _make_sc_triplet_agg():
    mesh = plsc.VectorSubcoreMesh(core_axis_name="c", subcore_axis_name="s")
    D = 64
    nv = D // 16
    ngrp = TCH // 16

    @functools.partial(
        pl.kernel, mesh=mesh,
        out_type=jax.ShapeDtypeStruct((E, D), jnp.float32),
        compiler_params=pltpu.CompilerParams(use_tc_tiling_on_sc=False),
        scratch_types=[
            pltpu.VMEM_SHARED((ACC, D), jnp.float32),
            pltpu.VMEM((32,), jnp.int32),
            pltpu.VMEM((TCH,), jnp.int32),
            pltpu.VMEM((TCH,), jnp.int32),
            pltpu.VMEM((TCH,), jnp.int32),
            pltpu.VMEM((TCH, D), jnp.float32),
            pltpu.VMEM((TCH, D), jnp.float32),
            pltpu.SemaphoreType.DMA,
        ],
    )
    def k(xd_hbm, ba_hbm, ca_hbm, cbf_hbm, off_hbm, zeros_hbm, out_hbm,
          acc_sh, off_v, bidx_v, cidx_v, idxl_v, rows_v, mul_v, sem):
        c = lax.axis_index("c")
        s = lax.axis_index("s")
        iot = lax.broadcasted_iota(jnp.int32, (16,), 0)

        def rnd(r, carry):
            k_blk = r * 2 + c
            ebase = k_blk * EBLK
            pltpu.sync_copy(zeros_hbm, acc_sh.at[pl.ds(s * 256, 256)])
            plsc.subcore_barrier()

            base8 = (k_blk // 8) * 8
            pltpu.sync_copy(off_hbm.at[pl.ds(base8, 32)], off_v)
            j = k_blk - base8
            wv = off_v[pl.ds(j, 16)]
            lo = wv[0]
            hi = wv[1]
            span = hi - lo
            lo_t = lo + (span * s) // 16
            hi_t = lo + (span * (s + 1)) // 16
            ta0 = (lo_t // 8) * 8
            iters = (hi_t - ta0 + TCH - 1) // TCH

            def chunk(jj, c2):
                ta = ta0 + jj * TCH
                pltpu.sync_copy(ba_hbm.at[pl.ds(ta, TCH)], bidx_v)
                pltpu.sync_copy(ca_hbm.at[pl.ds(ta, TCH)], cidx_v)
                pltpu.sync_copy(cbf_hbm.at[pl.ds(ta, TCH)], mul_v)
                pltpu.async_copy(xd_hbm.at[bidx_v], rows_v, sem).wait()

                def mulrow(i, c3):
                    for v in range(nv):
                        sl = pl.ds(v * 16, 16)
                        rows_v[i, sl] = rows_v[i, sl] * mul_v[i, sl]
                    return c3

                lax.fori_loop(0, TCH, mulrow, 0)

                def grp(g, c3):
                    sl = pl.ds(g * 16, 16)
                    ids = cidx_v[sl]
                    gl = ta + g * 16 + iot
                    valid = (gl >= lo_t) & (gl < hi_t)
                    idxl_v[sl] = jnp.where(valid, ids - ebase, EBLK + iot)
                    return c3

                lax.fori_loop(0, ngrp, grp, 0)
                pltpu.sync_copy(rows_v, acc_sh.at[idxl_v], add=True)
                return c2

            lax.fori_loop(0, iters, chunk, 0)
            plsc.subcore_barrier()

            @pl.when(s < 15)
            def _():
                pltpu.sync_copy(
                    acc_sh.at[pl.ds(s * 256, 256)],
                    out_hbm.at[pl.ds(ebase + s * 256, 256)])

            @pl.when(s == 15)
            def _():
                pltpu.sync_copy(
                    acc_sh.at[pl.ds(3840, EBLK - 3840)],
                    out_hbm.at[pl.ds(ebase + 3840, EBLK - 3840)])

            plsc.subcore_barrier()
            return carry

        lax.fori_loop(0, NROUND, rnd, 0)

    return k


def _triplet_agg(xd, ba_pad, ca_pad, cbf, offs):
    zeros = jnp.zeros((256, 64), jnp.float32)
    return _make_sc_triplet_agg()(xd, ba_pad, ca_pad, cbf, offs, zeros)


def _gather(table, idx):
    R, D = table.shape
    B = idx.shape[0]
    b_per_w = B // _NW
    budget = min(2000, (400000 // (D * 4 + 4)) // 8 * 8)
    chunk = 8
    for c in range(budget, 7, -8):
        if b_per_w % c == 0 and c % 8 == 0:
            chunk = c
            break
    return _make_sc_gather(R, D, B, chunk)(table, idx)


@functools.cache
def _make_sc_scatter_add(S, D, B, chunk):
    b_per_c = B // 2
    b_per_w = B // _NW
    iters = b_per_w // chunk
    stripe = S // 16
    assert b_per_w % chunk == 0 and chunk % 8 == 0 and S % 16 == 0
    assert stripe % 8 == 0
    mesh = plsc.VectorSubcoreMesh(core_axis_name="c", subcore_axis_name="s")

    @functools.partial(
        pl.kernel, mesh=mesh,
        out_type=jax.ShapeDtypeStruct((2, S, D), jnp.float32),
        compiler_params=pltpu.CompilerParams(use_tc_tiling_on_sc=False),
        scratch_types=[
            pltpu.VMEM_SHARED((S, D), jnp.float32),
            pltpu.VMEM((chunk,), jnp.int32),
            pltpu.VMEM((chunk, D), jnp.float32),
        ],
    )
    def k(rows_hbm, idx_hbm, zeros_hbm, out_hbm, acc_sh, idx_v, rows_v):
        c = lax.axis_index("c")
        s = lax.axis_index("s")
        pltpu.sync_copy(zeros_hbm, acc_sh.at[pl.ds(s * stripe, stripe)])
        plsc.subcore_barrier()
        base = c * b_per_c + s * b_per_w

        def body(j, carry):
            off = base + j * chunk
            pltpu.sync_copy(idx_hbm.at[pl.ds(off, chunk)], idx_v)
            pltpu.sync_copy(rows_hbm.at[pl.ds(off, chunk)], rows_v)
            pltpu.sync_copy(rows_v, acc_sh.at[idx_v], add=True)
            return carry

        lax.fori_loop(0, iters, body, 0)
        plsc.subcore_barrier()
        pltpu.sync_copy(acc_sh.at[pl.ds(s * stripe, stripe)],
                        out_hbm.at[c, pl.ds(s * stripe, stripe)])

    return k


def _segsum_n(rows, idx):
    B, D = rows.shape
    b_per_w = B // _NW
    budget = (400000 // (D * 4 + 4)) // 8 * 8
    spare = (1900000 - NPAD * D) // (16 * D) // 8 * 8
    budget = min(budget, spare)
    chunk = 8
    for c in range(budget, 7, -8):
        if b_per_w % c == 0:
            chunk = c
            break
    zeros = jnp.zeros((NPAD // 16, D), jnp.float32)
    return _make_sc_scatter_add(NPAD, D, B, chunk)(rows, idx, zeros)


def kernel(pos, latent, atom_table, Wl1, Wl2, W_edge, W_rbf3, W_rbf_h,
           W_rbf_out, W_rbf_cbf, W_sph, W_cbf_up, W1s, Wdowns, Wups, Wgs,
           Whs, Was, Wes, Wout_e, Wout_f, z, edge_index, id3_ba, id3_ca):
    idx_s = edge_index[0].astype(jnp.int32)
    idx_t = edge_index[1].astype(jnp.int32)
    ca32 = id3_ca.astype(jnp.int32)
    ba = jnp.zeros((TP,), jnp.int32).at[:T].set(id3_ba.astype(jnp.int32))
    ca = jnp.zeros((TP,), jnp.int32).at[:T].set(ca32)
    offs = jnp.zeros((128,), jnp.int32).at[:E // EBLK + 1].set(
        jnp.searchsorted(
            ca32, jnp.arange(0, E + 1, EBLK, dtype=jnp.int32)
        ).astype(jnp.int32))

    pos16 = jnp.zeros((N, 16), jnp.float32).at[:, :3].set(pos)
    W_rbf_all = jnp.concatenate([W_rbf3, W_rbf_h, W_rbf_out, W_rbf_cbf], axis=1)

    h = _node_init(z, latent, atom_table, Wl1, Wl2)

    pos_s = _gather(pos16, idx_s)
    pos_t = _gather(pos16, idx_t)
    h_s = _gather(h, idx_s)
    h_t = _gather(h, idx_t)
    m, unit, rbfp = _edge_stage(pos_s, pos_t, h_s, h_t, W_edge, W_rbf_all)

    rbfc = rbfp[:, 48:64] + 0.0
    unit_ba = _gather(unit, ba)
    unit_ca = _gather(unit, ca)
    rbfc_ca = _gather(rbfc, ca)
    cbf = _cbf_stage(unit_ba, unit_ca, rbfc_ca, W_sph, W_cbf_up)

    for b in range(NB):
        xd = _down_stage(m, W1s[b], Wdowns[b])
        agg = _triplet_agg(xd, ba, ca, cbf, offs)
        m, am = _mup_stage(m, agg, rbfp, Wups[b], Wgs[b], Whs[b])
        ha2 = _segsum_n(am, idx_t)
        h = _hup_stage(h, ha2, Was[b])
        h_s = _gather(h, idx_s)
        h_t = _gather(h, idx_t)
        m = _eres_stage(m, h_s, h_t, Wes[b])

    fu = _fout_stage(m, rbfp, unit, Wout_f)
    F2 = _segsum_n(fu, idx_t)
    ea, F = _nout_stage(h, F2, Wout_e)
    return jnp.concatenate([ea[:, 0:1], F[:, 0:3]], axis=1)
